# Initial kernel scaffold; baseline (speedup 1.0000x reference)
import jax
import jax.numpy as jnp
from jax.experimental import pallas as pl


def kernel(preds, targets_boxes, targets_labels):
    raise NotImplementedError("write your pallas kernel here")


# speedup vs baseline: 63.9925x; 63.9925x over previous
import functools
import math

import jax
import jax.numpy as jnp
from jax.experimental import pallas as pl
from jax.experimental.pallas import tpu as pltpu

NUM_CLASSES = 3
IMG_SIZE = 640.0
LAMBDA_BOX = 7.5
LAMBDA_CLS = 0.5
TOPK = 10
ALPHA_COST = 0.5
BETA_COST = 6.0
GAMMA = 2.0
ALPHA = 0.25
EPS = 1e-6
CLASS_WEIGHTS = (1.1757211179195934, 0.09527723808100434, 1.7290016439994023)

N_ANCH = 4800
N_GT = 20
S = 40
STRIDE = IMG_SIZE / S
BIG = 1e30


def _sigmoid(x):
    return 1.0 / (1.0 + jnp.exp(-x))


def _softplus(x):
    return jnp.maximum(x, 0.0) + jnp.log1p(jnp.exp(-jnp.abs(x)))


def _atan_pos(r):
    z = jnp.minimum(r, 1.0 / r)
    t = z / (1.0 + jnp.sqrt(1.0 + z * z))
    t2 = t * t
    p = t * (1.0 + t2 * (-1.0 / 3.0 + t2 * (1.0 / 5.0 + t2 * (-1.0 / 7.0
         + t2 * (1.0 / 9.0 + t2 * (-1.0 / 11.0 + t2 * (1.0 / 13.0
         + t2 * (-1.0 / 15.0))))))))
    a = 2.0 * p
    return jnp.where(r <= 1.0, a, math.pi / 2.0 - a)


def _loss_kernel(p_ref, tb_ref, lab_ref, out_ref,
                 alive_ref, aopen_ref, gtopen_ref, ai_ref):
    p = p_ref[0]
    tx = p[0:1, :]
    ty = p[1:2, :]
    tw = p[2:3, :]
    th = p[3:4, :]
    l0 = p[4:5, :]
    l1 = p[5:6, :]
    l2 = p[6:7, :]

    lane = jax.lax.broadcasted_iota(jnp.int32, (1, N_ANCH), 1)
    gx = (lane % S).astype(jnp.float32)
    gy = ((lane // S) % S).astype(jnp.float32)

    px = (_sigmoid(tx) * 2.0 - 0.5 + gx) * STRIDE
    py = (_sigmoid(ty) * 2.0 - 0.5 + gy) * STRIDE
    pw = _softplus(tw) * STRIDE
    ph = _softplus(th) * STRIDE
    px1 = px - 0.5 * pw
    py1 = py - 0.5 * ph
    px2 = px + 0.5 * pw
    py2 = py + 0.5 * ph

    t = tb_ref[0] * IMG_SIZE
    tx1 = t[:, 0:1]
    ty1 = t[:, 1:2]
    tx2 = t[:, 2:3]
    ty2 = t[:, 3:4]

    labels = lab_ref[0]
    labcol = labels.reshape(N_GT, 1)

    p_sel = jnp.zeros((N_GT, N_ANCH), dtype=jnp.float32)
    for c, lc in enumerate((l0, l1, l2)):
        mc = (labcol == c).astype(jnp.float32)
        p_sel = p_sel + mc * _sigmoid(lc)
    cost_cls = -jnp.log(jnp.clip(p_sel, EPS, 1.0 - EPS))

    e7 = 1e-7
    parea = jnp.maximum(px2 - px1, 0.0) * jnp.maximum(py2 - py1, 0.0)
    tarea = jnp.maximum(tx2 - tx1, 0.0) * jnp.maximum(ty2 - ty1, 0.0)
    ix1 = jnp.maximum(px1, tx1)
    iy1 = jnp.maximum(py1, ty1)
    ix2 = jnp.minimum(px2, tx2)
    iy2 = jnp.minimum(py2, ty2)
    inter = jnp.maximum(ix2 - ix1, 0.0) * jnp.maximum(iy2 - iy1, 0.0)
    union = parea + tarea - inter + e7
    iou = jnp.clip(inter / union, 0.0, 1.0)

    cost = ALPHA_COST * cost_cls + BETA_COST * (1.0 - iou)

    lane_f = lane
    grow = jax.lax.broadcasted_iota(jnp.int32, (N_GT, 1), 0)

    alive_ref[...] = jnp.zeros((N_GT, N_ANCH), dtype=jnp.float32)
    aopen_ref[...] = jnp.zeros((1, N_ANCH), dtype=jnp.float32)

    def topk_body(_, carry):
        dead = alive_ref[...] > 0.5
        vals = jnp.where(dead, BIG, cost)
        m = jnp.min(vals, axis=1, keepdims=True)
        isman = vals <= m
        nidx = jnp.min(jnp.where(isman, lane_f, N_ANCH), axis=1, keepdims=True)
        sel = (lane_f == nidx).astype(jnp.float32)
        alive_ref[...] = alive_ref[...] + sel
        aopen_ref[...] = jnp.maximum(aopen_ref[...],
                                     jnp.max(sel, axis=0, keepdims=True))
        return carry

    jax.lax.fori_loop(0, TOPK, topk_body, 0)

    gtopen_ref[...] = jnp.ones((N_GT, 1), dtype=jnp.float32)
    ai_ref[...] = jnp.full((N_GT, 1), -1, dtype=jnp.int32)

    def greedy_body(_, carry):
        valid = (gtopen_ref[...] > 0.5) & (aopen_ref[...] > 0.5)
        vals = jnp.where(valid, cost, BIG)
        m = jnp.min(vals)
        ok = m < BIG * 0.5
        hit = (vals <= m) & ok
        nstar = jnp.min(jnp.where(hit, lane_f, N_ANCH))
        hitcol = hit & (lane_f == nstar)
        gstar = jnp.min(jnp.where(hitcol, grow, N_GT))
        gsel = (grow == gstar) & ok
        ai_ref[...] = jnp.where(gsel, nstar, ai_ref[...])
        gtopen_ref[...] = jnp.where(gsel, 0.0, gtopen_ref[...])
        aopen_ref[...] = jnp.where((lane_f == nstar) & ok, 0.0, aopen_ref[...])
        return carry

    jax.lax.fori_loop(0, N_GT, greedy_body, 0)

    ai = ai_ref[...]
    validg = ai >= 0
    validf = validg.astype(jnp.float32)
    aidx = jnp.where(validg, ai, 0)

    onehot = (lane_f == aidx).astype(jnp.float32)
    pbx1 = jnp.sum(onehot * px1, axis=1, keepdims=True)
    pby1 = jnp.sum(onehot * py1, axis=1, keepdims=True)
    pbx2 = jnp.sum(onehot * px2, axis=1, keepdims=True)
    pby2 = jnp.sum(onehot * py2, axis=1, keepdims=True)

    pwg = jnp.maximum(pbx2 - pbx1, e7)
    phg = jnp.maximum(pby2 - pby1, e7)
    twg = jnp.maximum(tx2 - tx1, e7)
    thg = jnp.maximum(ty2 - ty1, e7)
    iw = jnp.maximum(jnp.minimum(pbx2, tx2) - jnp.maximum(pbx1, tx1), 0.0)
    ih = jnp.maximum(jnp.minimum(pby2, ty2) - jnp.maximum(pby1, ty1), 0.0)
    inter_g = iw * ih
    union_g = pwg * phg + twg * thg - inter_g + e7
    iou_g = inter_g / union_g
    cw = jnp.maximum(pbx2, tx2) - jnp.minimum(pbx1, tx1)
    ch = jnp.maximum(pby2, ty2) - jnp.minimum(pby1, ty1)
    c2 = cw * cw + ch * ch + e7
    rho2 = ((pbx1 + pbx2 - tx1 - tx2) ** 2 + (pby1 + pby2 - ty1 - ty2) ** 2) / 4.0
    datan = _atan_pos(twg / thg) - _atan_pos(pwg / phg)
    v = (4.0 / (math.pi ** 2)) * datan * datan
    alpha_t = v / (v - iou_g + 1.0 + e7)
    ciou = iou_g - rho2 / c2 - alpha_t * v
    loss_box = jnp.sum((1.0 - ciou) * validf)

    loss_cls = jnp.zeros((), dtype=jnp.float32)
    for c, lc in enumerate((l0, l1, l2)):
        wc = validf * (labcol == c).astype(jnp.float32)
        tcls = jnp.minimum(jnp.sum(onehot * wc, axis=0, keepdims=True), 1.0)
        probs = _sigmoid(lc)
        ce = jnp.maximum(lc, 0.0) - lc * tcls + jnp.log1p(jnp.exp(-jnp.abs(lc)))
        p_t = probs * tcls + (1.0 - probs) * (1.0 - tcls)
        fl = ce * (1.0 - p_t) ** GAMMA
        alpha_w = ALPHA * tcls + (1.0 - ALPHA) * (1.0 - tcls)
        loss_cls = loss_cls + CLASS_WEIGHTS[c] * jnp.sum(alpha_w * fl)

    npos = jnp.sum(validf)
    lane128 = jax.lax.broadcasted_iota(jnp.int32, (1, 128), 1)
    outvec = (jnp.where(lane128 == 0, loss_box, 0.0)
              + jnp.where(lane128 == 1, loss_cls, 0.0)
              + jnp.where(lane128 == 2, npos, 0.0))
    out_ref[0] = outvec


@jax.jit
def kernel(preds, targets_boxes, targets_labels):
    B = preds.shape[0]
    p = preds.reshape(B, N_ANCH, 7).transpose(0, 2, 1)
    tb = targets_boxes.astype(jnp.float32)
    lab = targets_labels.astype(jnp.int32).reshape(B, 1, N_GT)

    out = pl.pallas_call(
        _loss_kernel,
        grid=(B,),
        in_specs=[
            pl.BlockSpec((1, 7, N_ANCH), lambda i: (i, 0, 0)),
            pl.BlockSpec((1, N_GT, 4), lambda i: (i, 0, 0)),
            pl.BlockSpec((1, 1, N_GT), lambda i: (i, 0, 0)),
        ],
        out_specs=pl.BlockSpec((1, 1, 128), lambda i: (i, 0, 0)),
        out_shape=jax.ShapeDtypeStruct((B, 1, 128), jnp.float32),
        scratch_shapes=[
            pltpu.VMEM((N_GT, N_ANCH), jnp.float32),
            pltpu.VMEM((1, N_ANCH), jnp.float32),
            pltpu.VMEM((N_GT, 1), jnp.float32),
            pltpu.VMEM((N_GT, 1), jnp.int32),
        ],
    )(p, tb, lab)

    loss_box = jnp.sum(out[:, 0, 0])
    loss_cls = jnp.sum(out[:, 0, 1])
    npos = jnp.sum(out[:, 0, 2])
    denom = jnp.maximum(npos, 1.0)
    return (LAMBDA_BOX * loss_box + LAMBDA_CLS * loss_cls) / denom


# speedup vs baseline: 68.8727x; 1.0763x over previous
import functools
import math

import jax
import jax.numpy as jnp
from jax.experimental import pallas as pl
from jax.experimental.pallas import tpu as pltpu

NUM_CLASSES = 3
IMG_SIZE = 640.0
LAMBDA_BOX = 7.5
LAMBDA_CLS = 0.5
TOPK = 10
ALPHA_COST = 0.5
BETA_COST = 6.0
GAMMA = 2.0
ALPHA = 0.25
EPS = 1e-6
CLASS_WEIGHTS = (1.1757211179195934, 0.09527723808100434, 1.7290016439994023)

N_ANCH = 4800
N_GT = 20
S = 40
STRIDE = IMG_SIZE / S
BIG = 1e30


def _sigmoid(x):
    return 1.0 / (1.0 + jnp.exp(-x))


def _softplus(x):
    return jnp.maximum(x, 0.0) + jnp.log1p(jnp.exp(-jnp.abs(x)))


def _atan_pos(r):
    z = jnp.minimum(r, 1.0 / r)
    t = z / (1.0 + jnp.sqrt(1.0 + z * z))
    t2 = t * t
    p = t * (1.0 + t2 * (-1.0 / 3.0 + t2 * (1.0 / 5.0 + t2 * (-1.0 / 7.0
         + t2 * (1.0 / 9.0 + t2 * (-1.0 / 11.0 + t2 * (1.0 / 13.0
         + t2 * (-1.0 / 15.0))))))))
    a = 2.0 * p
    return jnp.where(r <= 1.0, a, math.pi / 2.0 - a)


def _loss_kernel(p_ref, tb_ref, lab_ref, out_ref):
    p = p_ref[0]
    tx = p[0:1, :]
    ty = p[1:2, :]
    tw = p[2:3, :]
    th = p[3:4, :]
    l0 = p[4:5, :]
    l1 = p[5:6, :]
    l2 = p[6:7, :]

    lane = jax.lax.broadcasted_iota(jnp.int32, (1, N_ANCH), 1)
    gx = (lane % S).astype(jnp.float32)
    gy = ((lane // S) % S).astype(jnp.float32)

    px = (_sigmoid(tx) * 2.0 - 0.5 + gx) * STRIDE
    py = (_sigmoid(ty) * 2.0 - 0.5 + gy) * STRIDE
    pw = _softplus(tw) * STRIDE
    ph = _softplus(th) * STRIDE
    px1 = px - 0.5 * pw
    py1 = py - 0.5 * ph
    px2 = px + 0.5 * pw
    py2 = py + 0.5 * ph

    t = tb_ref[0] * IMG_SIZE
    tx1 = t[:, 0:1]
    ty1 = t[:, 1:2]
    tx2 = t[:, 2:3]
    ty2 = t[:, 3:4]

    labels = lab_ref[0]
    labcol = labels.reshape(N_GT, 1)

    p_sel = jnp.zeros((N_GT, N_ANCH), dtype=jnp.float32)
    for c, lc in enumerate((l0, l1, l2)):
        mc = (labcol == c).astype(jnp.float32)
        p_sel = p_sel + mc * _sigmoid(lc)
    cost_cls = -jnp.log(jnp.clip(p_sel, EPS, 1.0 - EPS))

    e7 = 1e-7
    parea = jnp.maximum(px2 - px1, 0.0) * jnp.maximum(py2 - py1, 0.0)
    tarea = jnp.maximum(tx2 - tx1, 0.0) * jnp.maximum(ty2 - ty1, 0.0)
    ix1 = jnp.maximum(px1, tx1)
    iy1 = jnp.maximum(py1, ty1)
    ix2 = jnp.minimum(px2, tx2)
    iy2 = jnp.minimum(py2, ty2)
    inter = jnp.maximum(ix2 - ix1, 0.0) * jnp.maximum(iy2 - iy1, 0.0)
    union = parea + tarea - inter + e7
    iou = jnp.clip(inter / union, 0.0, 1.0)

    cost = ALPHA_COST * cost_cls + BETA_COST * (1.0 - iou)

    lane_f = lane
    grow = jax.lax.broadcasted_iota(jnp.int32, (N_GT, 1), 0)

    lane_f32 = lane.astype(jnp.float32)
    dn = (((1,), (1,)), ((), ()))
    bump = jnp.zeros((N_GT, N_ANCH), dtype=jnp.float32)
    subs = []
    cands = []
    for _ in range(TOPK):
        vals = cost + bump
        m = jnp.min(vals, axis=1, keepdims=True)
        nidx = jnp.min(jnp.where(vals <= m, lane_f, N_ANCH),
                       axis=1, keepdims=True)
        sel = (lane_f == nidx).astype(jnp.float32)
        bump = bump + sel * BIG
        subs.append(jax.lax.dot_general(
            cost, sel, dn, precision=jax.lax.Precision.HIGHEST,
            preferred_element_type=jnp.float32))
        cands.append(jax.lax.dot_general(
            lane_f32, sel, dn, precision=jax.lax.Precision.HIGHEST,
            preferred_element_type=jnp.float32))

    M = jnp.concatenate(subs, axis=1)
    candf = jnp.concatenate(cands, axis=1)
    NC = N_GT * TOPK

    lane_c = jax.lax.broadcasted_iota(jnp.int32, (1, NC), 1)
    ai = jnp.full((N_GT, 1), -1, dtype=jnp.int32)
    for _ in range(N_GT):
        m = jnp.min(M)
        ok = m < BIG * 0.5
        hit = (M <= m) & ok
        cstar = jnp.min(jnp.where(hit, lane_c, NC))
        hitcol = hit & (lane_c == cstar)
        gstar = jnp.min(jnp.where(hitcol, grow, N_GT))
        astar = jnp.min(jnp.where(lane_c == cstar, candf, BIG))
        gsel = (grow == gstar) & ok
        ai = jnp.where(gsel, astar.astype(jnp.int32), ai)
        M = (M + gsel.astype(jnp.float32) * BIG
             + ((candf == astar) & ok).astype(jnp.float32) * BIG)

    validg = ai >= 0
    validf = validg.astype(jnp.float32)
    aidx = jnp.where(validg, ai, 0)

    onehot = (lane_f == aidx).astype(jnp.float32)
    pbx1 = jnp.sum(onehot * px1, axis=1, keepdims=True)
    pby1 = jnp.sum(onehot * py1, axis=1, keepdims=True)
    pbx2 = jnp.sum(onehot * px2, axis=1, keepdims=True)
    pby2 = jnp.sum(onehot * py2, axis=1, keepdims=True)

    pwg = jnp.maximum(pbx2 - pbx1, e7)
    phg = jnp.maximum(pby2 - pby1, e7)
    twg = jnp.maximum(tx2 - tx1, e7)
    thg = jnp.maximum(ty2 - ty1, e7)
    iw = jnp.maximum(jnp.minimum(pbx2, tx2) - jnp.maximum(pbx1, tx1), 0.0)
    ih = jnp.maximum(jnp.minimum(pby2, ty2) - jnp.maximum(pby1, ty1), 0.0)
    inter_g = iw * ih
    union_g = pwg * phg + twg * thg - inter_g + e7
    iou_g = inter_g / union_g
    cw = jnp.maximum(pbx2, tx2) - jnp.minimum(pbx1, tx1)
    ch = jnp.maximum(pby2, ty2) - jnp.minimum(pby1, ty1)
    c2 = cw * cw + ch * ch + e7
    rho2 = ((pbx1 + pbx2 - tx1 - tx2) ** 2 + (pby1 + pby2 - ty1 - ty2) ** 2) / 4.0
    datan = _atan_pos(twg / thg) - _atan_pos(pwg / phg)
    v = (4.0 / (math.pi ** 2)) * datan * datan
    alpha_t = v / (v - iou_g + 1.0 + e7)
    ciou = iou_g - rho2 / c2 - alpha_t * v
    loss_box = jnp.sum((1.0 - ciou) * validf)

    loss_cls = jnp.zeros((), dtype=jnp.float32)
    for c, lc in enumerate((l0, l1, l2)):
        wc = validf * (labcol == c).astype(jnp.float32)
        tcls = jnp.minimum(jnp.sum(onehot * wc, axis=0, keepdims=True), 1.0)
        probs = _sigmoid(lc)
        ce = jnp.maximum(lc, 0.0) - lc * tcls + jnp.log1p(jnp.exp(-jnp.abs(lc)))
        p_t = probs * tcls + (1.0 - probs) * (1.0 - tcls)
        fl = ce * (1.0 - p_t) ** GAMMA
        alpha_w = ALPHA * tcls + (1.0 - ALPHA) * (1.0 - tcls)
        loss_cls = loss_cls + CLASS_WEIGHTS[c] * jnp.sum(alpha_w * fl)

    npos = jnp.sum(validf)
    lane128 = jax.lax.broadcasted_iota(jnp.int32, (1, 128), 1)
    outvec = (jnp.where(lane128 == 0, loss_box, 0.0)
              + jnp.where(lane128 == 1, loss_cls, 0.0)
              + jnp.where(lane128 == 2, npos, 0.0))
    out_ref[0] = outvec


@jax.jit
def kernel(preds, targets_boxes, targets_labels):
    B = preds.shape[0]
    p = preds.reshape(B, N_ANCH, 7).transpose(0, 2, 1)
    tb = targets_boxes.astype(jnp.float32)
    lab = targets_labels.astype(jnp.int32).reshape(B, 1, N_GT)

    out = pl.pallas_call(
        _loss_kernel,
        grid=(B,),
        in_specs=[
            pl.BlockSpec((1, 7, N_ANCH), lambda i: (i, 0, 0)),
            pl.BlockSpec((1, N_GT, 4), lambda i: (i, 0, 0)),
            pl.BlockSpec((1, 1, N_GT), lambda i: (i, 0, 0)),
        ],
        out_specs=pl.BlockSpec((1, 1, 128), lambda i: (i, 0, 0)),
        out_shape=jax.ShapeDtypeStruct((B, 1, 128), jnp.float32),
    )(p, tb, lab)

    loss_box = jnp.sum(out[:, 0, 0])
    loss_cls = jnp.sum(out[:, 0, 1])
    npos = jnp.sum(out[:, 0, 2])
    denom = jnp.maximum(npos, 1.0)
    return (LAMBDA_BOX * loss_box + LAMBDA_CLS * loss_cls) / denom
